# Initial kernel scaffold; baseline (speedup 1.0000x reference)
import jax
import jax.numpy as jnp
from jax.experimental import pallas as pl


def kernel(x, pos, edge_index, W1, b1, W2, b2, W3, b3):
    raise NotImplementedError("write your pallas kernel here")


# speedup vs baseline: 3.2932x; 3.2932x over previous
import functools

import jax
import jax.numpy as jnp
from jax import lax
from jax.experimental import pallas as pl
from jax.experimental.pallas import tpu as pltpu
from jax.experimental.pallas import tpu_sc as plsc

N = 10000
E = 320000
H = 128
TW = 144
PW = 16

NC = 2
NS = 16
NWK = NC * NS
EPW = E // NWK
KG = 80
NCG = EPW // KG
KS = 80
NCS = EPW // KS
NP = 10240
SLAB = NP // NS
CW = 16

BN = 2000
BE = 2000


def _prep_body(x_ref, pp_ref, w1at_ref, w1bt_ref, b1_ref, a_ref, b_ref):
    xb = x_ref[...]
    pp = pp_ref[...]
    a = jnp.dot(xb, w1at_ref[...], preferred_element_type=jnp.float32) + b1_ref[...]
    b = jnp.dot(xb, w1bt_ref[...], preferred_element_type=jnp.float32)
    a_ref[...] = jnp.concatenate([a, pp], axis=1)
    b_ref[...] = jnp.concatenate([b, pp], axis=1)


def _prep(x, pos_pad, w1at, w1bt, b1r):
    grid = (N // BN,)
    return pl.pallas_call(
        _prep_body,
        grid=grid,
        in_specs=[
            pl.BlockSpec((BN, H), lambda i: (i, 0)),
            pl.BlockSpec((BN, PW), lambda i: (i, 0)),
            pl.BlockSpec((H, H), lambda i: (0, 0)),
            pl.BlockSpec((H, H), lambda i: (0, 0)),
            pl.BlockSpec((1, H), lambda i: (0, 0)),
        ],
        out_specs=[
            pl.BlockSpec((BN, TW), lambda i: (i, 0)),
            pl.BlockSpec((BN, TW), lambda i: (i, 0)),
        ],
        out_shape=[
            jax.ShapeDtypeStruct((N, TW), jnp.float32),
            jax.ShapeDtypeStruct((N, TW), jnp.float32),
        ],
    )(x, pos_pad, w1at, w1bt, b1r)


def _gather_body(acat, bcat, col_h, row_h, hout,
                 colv, rowv, ga, gb, hp, sem_a, sem_b):
    wid = lax.axis_index("s") * NC + lax.axis_index("c")

    def chunk(ci, carry):
        base = wid * EPW + ci * KG
        pltpu.sync_copy(col_h.at[pl.ds(base, KG)], colv)
        pltpu.sync_copy(row_h.at[pl.ds(base, KG)], rowv)
        ca = pltpu.async_copy(acat.at[colv], ga, sem_a)
        cb = pltpu.async_copy(bcat.at[rowv], gb, sem_b)
        ca.wait()
        cb.wait()

        def edge(e, c2):
            for g in range(8):
                sl = pl.ds(16 * g, 16)
                hp[e, sl] = ga[e, sl] + gb[e, sl]
            rel = ga[e, pl.ds(H, 16)] - gb[e, pl.ds(H, 16)]
            d2 = rel[0] * rel[0] + rel[1] * rel[1] + rel[2] * rel[2]
            hp[e, pl.ds(H, 16)] = jnp.full((16,), d2, jnp.float32)
            return c2

        lax.fori_loop(0, KG, edge, 0)
        pltpu.sync_copy(hp, hout.at[pl.ds(base, KG)])
        return carry

    lax.fori_loop(0, NCG, chunk, 0)


def _gather(acat, bcat, col, row):
    mesh = plsc.VectorSubcoreMesh(
        core_axis_name="c", subcore_axis_name="s", num_cores=NC, num_subcores=NS)
    return pl.kernel(
        _gather_body,
        out_type=jax.ShapeDtypeStruct((E, TW), jnp.float32),
        mesh=mesh,
        compiler_params=pltpu.CompilerParams(use_tc_tiling_on_sc=False),
        scratch_types=[
            pltpu.VMEM((KG,), jnp.int32),
            pltpu.VMEM((KG,), jnp.int32),
            pltpu.VMEM((KG, TW), jnp.float32),
            pltpu.VMEM((KG, TW), jnp.float32),
            pltpu.VMEM((KG, TW), jnp.float32),
            pltpu.SemaphoreType.DMA,
            pltpu.SemaphoreType.DMA,
        ],
    )(acat, bcat, col, row)


def _edge_body(hc_ref, w1c_ref, w2t_ref, b2_ref, m_ref):
    hc = hc_ref[...]
    hpre = hc[:, :H]
    d2 = hc[:, H:H + 1]
    dist = jnp.sqrt(d2)
    h = jnp.maximum(hpre + dist * w1c_ref[...], 0.0)
    m = jnp.dot(h, w2t_ref[...], preferred_element_type=jnp.float32) + b2_ref[...]
    m_ref[...] = jnp.maximum(m, 0.0)


def _edge_mlp(hcat, w1cr, w2t, b2r):
    grid = (E // BE,)
    return pl.pallas_call(
        _edge_body,
        grid=grid,
        in_specs=[
            pl.BlockSpec((BE, TW), lambda i: (i, 0)),
            pl.BlockSpec((1, H), lambda i: (0, 0)),
            pl.BlockSpec((H, H), lambda i: (0, 0)),
            pl.BlockSpec((1, H), lambda i: (0, 0)),
        ],
        out_specs=pl.BlockSpec((BE, H), lambda i: (i, 0)),
        out_shape=jax.ShapeDtypeStruct((E, H), jnp.float32),
    )(hcat, w1cr, w2t, b2r)


def _scatter_body(m_h, row_h, zer_h, zer8_h, one8_h,
                  s0, s1, c0, c1,
                  idxv, mv, onesv, ssum, scnt):
    cid = lax.axis_index("c")
    sid = lax.axis_index("s")
    wid = sid * NC + cid

    slab = pl.ds(sid * SLAB, SLAB)
    pltpu.sync_copy(zer_h, ssum.at[slab])
    pltpu.sync_copy(zer8_h, scnt.at[slab])
    pltpu.sync_copy(one8_h, onesv)
    plsc.subcore_barrier()

    def chunk(ci, carry):
        base = wid * EPW + ci * KS
        pltpu.sync_copy(row_h.at[pl.ds(base, KS)], idxv)
        pltpu.sync_copy(m_h.at[pl.ds(base, KS)], mv)
        pltpu.sync_copy(mv, ssum.at[idxv], add=True)
        pltpu.sync_copy(onesv, scnt.at[idxv], add=True)
        return carry

    lax.fori_loop(0, NCS, chunk, 0)
    plsc.subcore_barrier()

    @pl.when(cid == 0)
    def _():
        pltpu.sync_copy(ssum.at[slab], s0.at[slab])
        pltpu.sync_copy(scnt.at[slab], c0.at[slab])

    @pl.when(cid == 1)
    def _():
        pltpu.sync_copy(ssum.at[slab], s1.at[slab])
        pltpu.sync_copy(scnt.at[slab], c1.at[slab])


def _scatter(m, row, zer, zer8, one8):
    mesh = plsc.VectorSubcoreMesh(
        core_axis_name="c", subcore_axis_name="s", num_cores=NC, num_subcores=NS)
    return pl.kernel(
        _scatter_body,
        out_type=[
            jax.ShapeDtypeStruct((NP, H), jnp.float32),
            jax.ShapeDtypeStruct((NP, H), jnp.float32),
            jax.ShapeDtypeStruct((NP, CW), jnp.float32),
            jax.ShapeDtypeStruct((NP, CW), jnp.float32),
        ],
        mesh=mesh,
        compiler_params=pltpu.CompilerParams(use_tc_tiling_on_sc=False),
        scratch_types=[
            pltpu.VMEM((KS,), jnp.int32),
            pltpu.VMEM((KS, H), jnp.float32),
            pltpu.VMEM((KS, CW), jnp.float32),
            pltpu.VMEM_SHARED((NP, H), jnp.float32),
            pltpu.VMEM_SHARED((NP, CW), jnp.float32),
        ],
    )(m, row, zer, zer8, one8)


def _final_body(x_ref, s0_ref, s1_ref, c0_ref, c1_ref, w3t_ref, b3_ref, o_ref):
    cnt = c0_ref[...][:, :1] + c1_ref[...][:, :1]
    m_i = (s0_ref[...] + s1_ref[...]) / jnp.clip(cnt, 1.0, None)
    cat = jnp.concatenate([x_ref[...], m_i], axis=1)
    o = jnp.dot(cat, w3t_ref[...], preferred_element_type=jnp.float32) + b3_ref[...]
    o_ref[...] = jnp.maximum(o, 0.0)


def _final(x, s0, s1, c0, c1, w3t, b3r):
    grid = (N // BN,)
    return pl.pallas_call(
        _final_body,
        grid=grid,
        in_specs=[
            pl.BlockSpec((BN, H), lambda i: (i, 0)),
            pl.BlockSpec((BN, H), lambda i: (i, 0)),
            pl.BlockSpec((BN, H), lambda i: (i, 0)),
            pl.BlockSpec((BN, CW), lambda i: (i, 0)),
            pl.BlockSpec((BN, CW), lambda i: (i, 0)),
            pl.BlockSpec((2 * H, H), lambda i: (0, 0)),
            pl.BlockSpec((1, H), lambda i: (0, 0)),
        ],
        out_specs=pl.BlockSpec((BN, H), lambda i: (i, 0)),
        out_shape=jax.ShapeDtypeStruct((N, H), jnp.float32),
    )(x, s0, s1, c0, c1, w3t, b3r)


@jax.jit
def kernel(x, pos, edge_index, W1, b1, W2, b2, W3, b3):
    ei = edge_index.astype(jnp.int32)
    row = ei[0]
    col = ei[1]

    w1at = W1[:, :H].T
    w1bt = W1[:, H:2 * H].T
    w1cr = W1[:, 2 * H].reshape(1, H)
    b1r = b1.reshape(1, H)
    w2t = W2.T
    b2r = b2.reshape(1, H)
    w3t = W3.T
    b3r = b3.reshape(1, H)

    pos_pad = jnp.concatenate(
        [pos, jnp.zeros((N, PW - 3), jnp.float32)], axis=1)

    zer = jnp.zeros((SLAB, H), jnp.float32)
    zer8 = jnp.zeros((SLAB, CW), jnp.float32)
    one8 = jnp.concatenate(
        [jnp.ones((KS, 1), jnp.float32), jnp.zeros((KS, CW - 1), jnp.float32)], axis=1)

    acat, bcat = _prep(x, pos_pad, w1at, w1bt, b1r)
    hcat = _gather(acat, bcat, col, row)
    m = _edge_mlp(hcat, w1cr, w2t, b2r)
    s0, s1, c0, c1 = _scatter(m, row, zer, zer8, one8)
    return _final(x, s0, s1, c0, c1, w3t, b3r)
